# baseline (speedup 1.0000x reference)
import jax
import jax.numpy as jnp
from jax import lax
from jax.experimental import pallas as pl
from jax.experimental.pallas import tpu as pltpu
from jax.experimental.pallas import tpu_sc as plsc

B = 1024
K1 = 256
D = 128
NROWS = 100000
MOM = 0.5

NC = 2
NS = 16
NW = NC * NS
GB = B * K1
GPW = GB // NW
CH = 128
NBUF = 4
LAG = 3
NCHUNK = GPW // CH
NGRP = NCHUNK // NBUF
assert NCHUNK % NBUF == 0 and GPW % CH == 0 and 0 < LAG < NBUF
UPW = B // NW
LANES = 16
DV = D // LANES

_mesh = plsc.VectorSubcoreMesh(core_axis_name="c", subcore_axis_name="s")


def _update_body(mem1, mem2, v1, v2, y, perm, w1dep, nm1, nm2,
                 yv, pv, mrow1, vrow1, urow1, mrow2, vrow2, urow2, usem):
  del w1dep
  wid = lax.axis_index("s") * NC + lax.axis_index("c")

  u0 = wid * UPW
  pltpu.sync_copy(y.at[pl.ds(u0, UPW)], yv)
  pltpu.sync_copy(perm.at[pl.ds(u0, UPW)], pv)

  cps = [pltpu.make_async_copy(mem1.at[yv], mrow1, usem),
         pltpu.make_async_copy(v1.at[pv], vrow1, usem),
         pltpu.make_async_copy(mem2.at[yv], mrow2, usem),
         pltpu.make_async_copy(v2.at[pv], vrow2, usem)]
  for cp in cps:
    cp.start()
  for cp in cps:
    cp.wait()

  def compute(mrow, vrow, urow):
    @pl.loop(0, UPW)
    def _row(r):
      ss = jnp.zeros((LANES,), jnp.float32)
      for j in range(DV):
        sl = pl.ds(j * LANES, LANES)
        lp = mrow[r, sl] * MOM + vrow[r, sl] * (1.0 - MOM)
        urow[r, sl] = lp
        ss = ss + lp * lp
      s = jnp.sum(ss)
      sv = jnp.full((LANES,), s, jnp.float32)
      iv = plsc.bitcast(sv, jnp.int32)
      iv = jnp.int32(0x5F3759DF) - (iv >> 1)
      inv = plsc.bitcast(iv, jnp.float32)
      for _ in range(3):
        inv = inv * (1.5 - 0.5 * sv * inv * inv)
      for j in range(DV):
        sl = pl.ds(j * LANES, LANES)
        urow[r, sl] = urow[r, sl] * inv

  compute(mrow1, vrow1, urow1)
  compute(mrow2, vrow2, urow2)
  sc1 = pltpu.make_async_copy(urow1, nm1.at[yv], usem)
  sc2 = pltpu.make_async_copy(urow2, nm2.at[yv], usem)
  sc1.start()
  sc2.start()
  sc1.wait()
  sc2.wait()


def _gather_body(mem1, mem2, fidx, w1, w2, idxv, rbuf, gsem, osem):
  wid = lax.axis_index("s") * NC + lax.axis_index("c")

  base = wid * GPW
  pltpu.sync_copy(fidx.at[pl.ds(base, GPW)], idxv)

  def big_gather(mem, wout):
    def gth(c, b):
      return pltpu.make_async_copy(
          mem.at[idxv.at[pl.ds(c * CH, CH)]], rbuf.at[b], gsem.at[b])

    def out(c, b):
      return pltpu.make_async_copy(
          rbuf.at[b], wout.at[pl.ds(base + c * CH, CH)], osem.at[b])

    for b in range(NBUF):
      gth(b, b).start()

    @pl.loop(0, NGRP)
    def _grp(g):
      c0 = g * NBUF
      for b in range(NBUF):
        c = c0 + b
        gth(c, b).wait()
        out(c, b).start()
        bl = (b - LAG) % NBUF
        cl = c - LAG

        @pl.when(cl >= 0)
        def _():
          out(cl, bl).wait()

          @pl.when(cl + NBUF < NCHUNK)
          def _():
            gth(cl + NBUF, bl).start()

    for t in range(LAG):
      c = NCHUNK - LAG + t
      out(c, c % NBUF).wait()

  big_gather(mem1, w1)
  big_gather(mem2, w2)


_gather_call = pl.kernel(
    _gather_body,
    out_type=[
        jax.ShapeDtypeStruct((GB, D), jnp.float32),
        jax.ShapeDtypeStruct((GB, D), jnp.float32),
    ],
    mesh=_mesh,
    compiler_params=pltpu.CompilerParams(needs_layout_passes=False),
    scratch_types=[
        pltpu.VMEM((GPW,), jnp.int32),
        pltpu.VMEM((NBUF, CH, D), jnp.float32),
        pltpu.SemaphoreType.DMA((NBUF,)),
        pltpu.SemaphoreType.DMA((NBUF,)),
    ],
)

_update_call = pl.kernel(
    _update_body,
    out_type=(),
    mesh=_mesh,
    compiler_params=pltpu.CompilerParams(needs_layout_passes=False),
    scratch_types=[
        pltpu.VMEM((UPW,), jnp.int32),
        pltpu.VMEM((UPW,), jnp.int32),
        pltpu.VMEM((UPW, D), jnp.float32),
        pltpu.VMEM((UPW, D), jnp.float32),
        pltpu.VMEM((UPW, D), jnp.float32),
        pltpu.VMEM((UPW, D), jnp.float32),
        pltpu.VMEM((UPW, D), jnp.float32),
        pltpu.VMEM((UPW, D), jnp.float32),
        pltpu.SemaphoreType.DMA,
    ],
)


def kernel(v1, v2, memory_v1, memory_v2, y, idx):
  fidx = idx.reshape(-1)
  jj = jnp.arange(B, dtype=jnp.int32)
  eq = y[:, None] == y[None, :]
  perm = jnp.max(jnp.where(eq, jj[None, :], jnp.int32(-1)), axis=1)
  nm1 = jax.new_ref(memory_v1)
  nm2 = jax.new_ref(memory_v2)
  w1, w2 = _gather_call(memory_v1, memory_v2, fidx)
  _update_call(memory_v1, memory_v2, v1, v2, y, perm, w1, nm1, nm2)
  return (w1.reshape(B, K1, D), w2.reshape(B, K1, D),
          jax.freeze(nm1), jax.freeze(nm2))


# speedup vs baseline: 1.1189x; 1.1189x over previous
import jax
import jax.numpy as jnp
from jax import lax
from jax.experimental import pallas as pl
from jax.experimental.pallas import tpu as pltpu
from jax.experimental.pallas import tpu_sc as plsc

B = 1024
K1 = 256
D = 128
NROWS = 100000
MOM = 0.5

NC = 2
NS = 16
NW = NC * NS
GB = B * K1
GPW = GB // NW
CH = 128
NBUF = 4
LAG = 2
NCHUNK = GPW // CH
NGRP = NCHUNK // NBUF
assert NCHUNK % NBUF == 0 and GPW % CH == 0 and 0 < LAG < NBUF
UPW = B // NW
LANES = 16
DV = D // LANES

_mesh = plsc.VectorSubcoreMesh(core_axis_name="c", subcore_axis_name="s")


def _update_body(mem1, mem2, v1, v2, y, perm, w1dep, nm1, nm2,
                 yv, pv, mrow1, vrow1, urow1, mrow2, vrow2, urow2, usem):
  del w1dep
  wid = lax.axis_index("s") * NC + lax.axis_index("c")

  u0 = wid * UPW
  pltpu.sync_copy(y.at[pl.ds(u0, UPW)], yv)
  pltpu.sync_copy(perm.at[pl.ds(u0, UPW)], pv)

  cps = [pltpu.make_async_copy(mem1.at[yv], mrow1, usem),
         pltpu.make_async_copy(v1.at[pv], vrow1, usem),
         pltpu.make_async_copy(mem2.at[yv], mrow2, usem),
         pltpu.make_async_copy(v2.at[pv], vrow2, usem)]
  for cp in cps:
    cp.start()
  for cp in cps:
    cp.wait()

  def compute(mrow, vrow, urow):
    @pl.loop(0, UPW)
    def _row(r):
      ss = jnp.zeros((LANES,), jnp.float32)
      for j in range(DV):
        sl = pl.ds(j * LANES, LANES)
        lp = mrow[r, sl] * MOM + vrow[r, sl] * (1.0 - MOM)
        urow[r, sl] = lp
        ss = ss + lp * lp
      s = jnp.sum(ss)
      sv = jnp.full((LANES,), s, jnp.float32)
      iv = plsc.bitcast(sv, jnp.int32)
      iv = jnp.int32(0x5F3759DF) - (iv >> 1)
      inv = plsc.bitcast(iv, jnp.float32)
      for _ in range(3):
        inv = inv * (1.5 - 0.5 * sv * inv * inv)
      for j in range(DV):
        sl = pl.ds(j * LANES, LANES)
        urow[r, sl] = urow[r, sl] * inv

  compute(mrow1, vrow1, urow1)
  compute(mrow2, vrow2, urow2)
  sc1 = pltpu.make_async_copy(urow1, nm1.at[yv], usem)
  sc2 = pltpu.make_async_copy(urow2, nm2.at[yv], usem)
  sc1.start()
  sc2.start()
  sc1.wait()
  sc2.wait()


def _gather_body(mem1, mem2, fidx, w1, w2, idxv, rbuf, gsem, osem):
  wid = lax.axis_index("s") * NC + lax.axis_index("c")

  base = wid * GPW
  pltpu.sync_copy(fidx.at[pl.ds(base, GPW)], idxv)

  def big_gather(mem, wout):
    def gth(c, b):
      return pltpu.make_async_copy(
          mem.at[idxv.at[pl.ds(c * CH, CH)]], rbuf.at[b], gsem.at[b])

    def out(c, b):
      return pltpu.make_async_copy(
          rbuf.at[b], wout.at[pl.ds(base + c * CH, CH)], osem.at[b])

    for b in range(NBUF):
      gth(b, b).start()

    @pl.loop(0, NGRP)
    def _grp(g):
      c0 = g * NBUF
      for b in range(NBUF):
        c = c0 + b
        gth(c, b).wait()
        out(c, b).start()
        bl = (b - LAG) % NBUF
        cl = c - LAG

        @pl.when(cl >= 0)
        def _():
          out(cl, bl).wait()

          @pl.when(cl + NBUF < NCHUNK)
          def _():
            gth(cl + NBUF, bl).start()

    for t in range(LAG):
      c = NCHUNK - LAG + t
      out(c, c % NBUF).wait()

  big_gather(mem1, w1)
  big_gather(mem2, w2)


_gather_call = pl.kernel(
    _gather_body,
    out_type=[
        jax.ShapeDtypeStruct((GB, D), jnp.float32),
        jax.ShapeDtypeStruct((GB, D), jnp.float32),
    ],
    mesh=_mesh,
    compiler_params=pltpu.CompilerParams(needs_layout_passes=False),
    scratch_types=[
        pltpu.VMEM((GPW,), jnp.int32),
        pltpu.VMEM((NBUF, CH, D), jnp.float32),
        pltpu.SemaphoreType.DMA((NBUF,)),
        pltpu.SemaphoreType.DMA((NBUF,)),
    ],
)

_update_call = pl.kernel(
    _update_body,
    out_type=(),
    mesh=_mesh,
    compiler_params=pltpu.CompilerParams(needs_layout_passes=False),
    scratch_types=[
        pltpu.VMEM((UPW,), jnp.int32),
        pltpu.VMEM((UPW,), jnp.int32),
        pltpu.VMEM((UPW, D), jnp.float32),
        pltpu.VMEM((UPW, D), jnp.float32),
        pltpu.VMEM((UPW, D), jnp.float32),
        pltpu.VMEM((UPW, D), jnp.float32),
        pltpu.VMEM((UPW, D), jnp.float32),
        pltpu.VMEM((UPW, D), jnp.float32),
        pltpu.SemaphoreType.DMA,
    ],
)


def kernel(v1, v2, memory_v1, memory_v2, y, idx):
  fidx = idx.reshape(-1)
  jj = jnp.arange(B, dtype=jnp.int32)
  eq = y[:, None] == y[None, :]
  perm = jnp.max(jnp.where(eq, jj[None, :], jnp.int32(-1)), axis=1)
  nm1 = jax.new_ref(memory_v1)
  nm2 = jax.new_ref(memory_v2)
  w1, w2 = _gather_call(memory_v1, memory_v2, fidx)
  _update_call(memory_v1, memory_v2, v1, v2, y, perm, w1, nm1, nm2)
  return (w1.reshape(B, K1, D), w2.reshape(B, K1, D),
          jax.freeze(nm1), jax.freeze(nm2))


# speedup vs baseline: 1.1222x; 1.0030x over previous
import jax
import jax.numpy as jnp
from jax import lax
from jax.experimental import pallas as pl
from jax.experimental.pallas import tpu as pltpu
from jax.experimental.pallas import tpu_sc as plsc

B = 1024
K1 = 256
D = 128
NROWS = 100000
MOM = 0.5

NC = 2
NS = 16
NW = NC * NS
GB = B * K1
GPW = GB // NW
CH = 64
NBUF = 8
LAG = 4
NCHUNK = GPW // CH
NGRP = NCHUNK // NBUF
assert NCHUNK % NBUF == 0 and GPW % CH == 0 and 0 < LAG < NBUF
UPW = B // NW
LANES = 16
DV = D // LANES

_mesh = plsc.VectorSubcoreMesh(core_axis_name="c", subcore_axis_name="s")


def _update_body(mem1, mem2, v1, v2, y, perm, w1dep, nm1, nm2,
                 yv, pv, mrow1, vrow1, urow1, mrow2, vrow2, urow2, usem):
  del w1dep
  wid = lax.axis_index("s") * NC + lax.axis_index("c")

  u0 = wid * UPW
  pltpu.sync_copy(y.at[pl.ds(u0, UPW)], yv)
  pltpu.sync_copy(perm.at[pl.ds(u0, UPW)], pv)

  cps = [pltpu.make_async_copy(mem1.at[yv], mrow1, usem),
         pltpu.make_async_copy(v1.at[pv], vrow1, usem),
         pltpu.make_async_copy(mem2.at[yv], mrow2, usem),
         pltpu.make_async_copy(v2.at[pv], vrow2, usem)]
  for cp in cps:
    cp.start()
  for cp in cps:
    cp.wait()

  def compute(mrow, vrow, urow):
    @pl.loop(0, UPW)
    def _row(r):
      ss = jnp.zeros((LANES,), jnp.float32)
      for j in range(DV):
        sl = pl.ds(j * LANES, LANES)
        lp = mrow[r, sl] * MOM + vrow[r, sl] * (1.0 - MOM)
        urow[r, sl] = lp
        ss = ss + lp * lp
      s = jnp.sum(ss)
      sv = jnp.full((LANES,), s, jnp.float32)
      iv = plsc.bitcast(sv, jnp.int32)
      iv = jnp.int32(0x5F3759DF) - (iv >> 1)
      inv = plsc.bitcast(iv, jnp.float32)
      for _ in range(3):
        inv = inv * (1.5 - 0.5 * sv * inv * inv)
      for j in range(DV):
        sl = pl.ds(j * LANES, LANES)
        urow[r, sl] = urow[r, sl] * inv

  compute(mrow1, vrow1, urow1)
  compute(mrow2, vrow2, urow2)
  sc1 = pltpu.make_async_copy(urow1, nm1.at[yv], usem)
  sc2 = pltpu.make_async_copy(urow2, nm2.at[yv], usem)
  sc1.start()
  sc2.start()
  sc1.wait()
  sc2.wait()


def _gather_body(mem1, mem2, fidx, w1, w2, idxv, rbuf, gsem, osem):
  wid = lax.axis_index("s") * NC + lax.axis_index("c")

  base = wid * GPW
  pltpu.sync_copy(fidx.at[pl.ds(base, GPW)], idxv)

  def big_gather(mem, wout):
    def gth(c, b):
      return pltpu.make_async_copy(
          mem.at[idxv.at[pl.ds(c * CH, CH)]], rbuf.at[b], gsem.at[b])

    def out(c, b):
      return pltpu.make_async_copy(
          rbuf.at[b], wout.at[pl.ds(base + c * CH, CH)], osem.at[b])

    for b in range(NBUF):
      gth(b, b).start()

    @pl.loop(0, NGRP)
    def _grp(g):
      c0 = g * NBUF
      for b in range(NBUF):
        c = c0 + b
        gth(c, b).wait()
        out(c, b).start()
        bl = (b - LAG) % NBUF
        cl = c - LAG

        @pl.when(cl >= 0)
        def _():
          out(cl, bl).wait()

          @pl.when(cl + NBUF < NCHUNK)
          def _():
            gth(cl + NBUF, bl).start()

    for t in range(LAG):
      c = NCHUNK - LAG + t
      out(c, c % NBUF).wait()

  big_gather(mem1, w1)
  big_gather(mem2, w2)


_gather_call = pl.kernel(
    _gather_body,
    out_type=[
        jax.ShapeDtypeStruct((GB, D), jnp.float32),
        jax.ShapeDtypeStruct((GB, D), jnp.float32),
    ],
    mesh=_mesh,
    compiler_params=pltpu.CompilerParams(needs_layout_passes=False),
    scratch_types=[
        pltpu.VMEM((GPW,), jnp.int32),
        pltpu.VMEM((NBUF, CH, D), jnp.float32),
        pltpu.SemaphoreType.DMA((NBUF,)),
        pltpu.SemaphoreType.DMA((NBUF,)),
    ],
)

_update_call = pl.kernel(
    _update_body,
    out_type=(),
    mesh=_mesh,
    compiler_params=pltpu.CompilerParams(needs_layout_passes=False),
    scratch_types=[
        pltpu.VMEM((UPW,), jnp.int32),
        pltpu.VMEM((UPW,), jnp.int32),
        pltpu.VMEM((UPW, D), jnp.float32),
        pltpu.VMEM((UPW, D), jnp.float32),
        pltpu.VMEM((UPW, D), jnp.float32),
        pltpu.VMEM((UPW, D), jnp.float32),
        pltpu.VMEM((UPW, D), jnp.float32),
        pltpu.VMEM((UPW, D), jnp.float32),
        pltpu.SemaphoreType.DMA,
    ],
)


def kernel(v1, v2, memory_v1, memory_v2, y, idx):
  fidx = idx.reshape(-1)
  jj = jnp.arange(B, dtype=jnp.int32)
  eq = y[:, None] == y[None, :]
  perm = jnp.max(jnp.where(eq, jj[None, :], jnp.int32(-1)), axis=1)
  nm1 = jax.new_ref(memory_v1)
  nm2 = jax.new_ref(memory_v2)
  w1, w2 = _gather_call(memory_v1, memory_v2, fidx)
  _update_call(memory_v1, memory_v2, v1, v2, y, perm, w1, nm1, nm2)
  return (w1.reshape(B, K1, D), w2.reshape(B, K1, D),
          jax.freeze(nm1), jax.freeze(nm2))
